# Initial kernel scaffold; baseline (speedup 1.0000x reference)
import jax
import jax.numpy as jnp
from jax.experimental import pallas as pl


def kernel(inputs, weight):
    raise NotImplementedError("write your pallas kernel here")


# speedup vs baseline: 1.1008x; 1.1008x over previous
import functools

import jax
import jax.numpy as jnp
from jax.experimental import pallas as pl
from jax.experimental.pallas import tpu as pltpu

_COMMITMENT_COST = 0.25


def _vq_kernel(x_ref, w_ref, enc_ref, q_ref, loss_ref, ppl_ref,
               acc_loss, acc_cnt, *, nblk, n_rows, n_codes, dim):
    i = pl.program_id(0)
    x = x_ref[...]
    w = w_ref[...]
    xsq = jnp.sum(x * x, axis=1, keepdims=True)
    wsq = jnp.sum(w * w, axis=1, keepdims=True).T
    mm = jax.lax.dot_general(
        x, w, (((1,), (1,)), ((), ())),
        preferred_element_type=jnp.float32)
    d = (xsq + wsq) - 2.0 * mm
    dmin = jnp.min(d, axis=1, keepdims=True)
    col = jax.lax.broadcasted_iota(jnp.int32, d.shape, 1)
    idx = jnp.min(jnp.where(d == dmin, col, n_codes),
                  axis=1, keepdims=True)
    enc = (col == idx).astype(jnp.float32)
    enc_ref[...] = enc
    q = jax.lax.dot_general(
        enc, w, (((1,), (0,)), ((), ())),
        preferred_element_type=jnp.float32)
    diff = q - x
    q_ref[...] = x + diff
    blk_loss = jnp.sum(diff * diff)
    cnt = jnp.sum(enc, axis=0, keepdims=True)

    @pl.when(i == 0)
    def _init():
        acc_loss[0, 0] = blk_loss
        acc_cnt[...] = cnt

    @pl.when(i > 0)
    def _acc():
        acc_loss[0, 0] += blk_loss
        acc_cnt[...] += cnt

    @pl.when(i == nblk - 1)
    def _fini():
        mean_sq = acc_loss[0, 0] / (n_rows * dim)
        loss_ref[...] = jnp.full((1, 1), mean_sq + _COMMITMENT_COST * mean_sq,
                                 jnp.float32)
        p = acc_cnt[...] / n_rows
        ent = jnp.sum(p * jnp.log(p + 1e-10), axis=1, keepdims=True)
        ppl_ref[...] = jnp.exp(-ent)


def kernel(inputs, weight):
    n, dim = inputs.shape
    n_codes = weight.shape[0]
    block_rows = 2048 if n % 2048 == 0 else n
    nblk = n // block_rows

    enc, q_ste, loss, ppl = pl.pallas_call(
        functools.partial(_vq_kernel, nblk=nblk, n_rows=n,
                          n_codes=n_codes, dim=dim),
        grid=(nblk,),
        in_specs=[
            pl.BlockSpec((block_rows, dim), lambda i: (i, 0)),
            pl.BlockSpec((n_codes, dim), lambda i: (0, 0)),
        ],
        out_specs=[
            pl.BlockSpec((block_rows, n_codes), lambda i: (i, 0)),
            pl.BlockSpec((block_rows, dim), lambda i: (i, 0)),
            pl.BlockSpec((1, 1), lambda i: (0, 0)),
            pl.BlockSpec((1, 1), lambda i: (0, 0)),
        ],
        out_shape=[
            jax.ShapeDtypeStruct((n, n_codes), jnp.float32),
            jax.ShapeDtypeStruct((n, dim), jnp.float32),
            jax.ShapeDtypeStruct((1, 1), jnp.float32),
            jax.ShapeDtypeStruct((1, 1), jnp.float32),
        ],
        scratch_shapes=[
            pltpu.SMEM((1, 1), jnp.float32),
            pltpu.VMEM((1, n_codes), jnp.float32),
        ],
    )(inputs, weight)

    return (loss.reshape(()), q_ste, ppl.reshape(()), enc)


# speedup vs baseline: 1.2034x; 1.0932x over previous
import functools

import jax
import jax.numpy as jnp
from jax.experimental import pallas as pl
from jax.experimental.pallas import tpu as pltpu

_COMMITMENT_COST = 0.25


def _vq_kernel(x_ref, w_ref, enc_ref, q_ref, loss_ref, ppl_ref,
               acc_loss, acc_cnt, *, nblk, n_rows, n_codes, dim):
    i = pl.program_id(0)
    x = x_ref[...]
    w = w_ref[...]
    xsq = jnp.sum(x * x, axis=1, keepdims=True)
    wsq = jnp.sum(w * w, axis=1, keepdims=True).T
    mm2 = jax.lax.dot_general(
        2.0 * x, w, (((1,), (1,)), ((), ())),
        preferred_element_type=jnp.float32)
    d = (xsq + wsq) - mm2
    dmin = jnp.min(d, axis=1, keepdims=True)
    col = jax.lax.broadcasted_iota(jnp.int32, d.shape, 1)
    idx = jnp.min(jnp.where(d == dmin, col, n_codes),
                  axis=1, keepdims=True)
    enc = (col == idx).astype(jnp.float32)
    enc_ref[...] = enc
    q = jax.lax.dot_general(
        enc, w, (((1,), (0,)), ((), ())),
        preferred_element_type=jnp.float32)
    diff = q - x
    q_ref[...] = x + diff
    blk_loss = jnp.sum(diff * diff)
    ones_row = jnp.ones((1, enc.shape[0]), jnp.float32)
    cnt = jax.lax.dot_general(
        ones_row, enc, (((1,), (0,)), ((), ())),
        preferred_element_type=jnp.float32)

    @pl.when(i == 0)
    def _init():
        acc_loss[0, 0] = blk_loss
        acc_cnt[...] = cnt

    @pl.when(i > 0)
    def _acc():
        acc_loss[0, 0] += blk_loss
        acc_cnt[...] += cnt

    @pl.when(i == nblk - 1)
    def _fini():
        mean_sq = acc_loss[0, 0] / (n_rows * dim)
        loss_ref[...] = jnp.full((1, 1), mean_sq + _COMMITMENT_COST * mean_sq,
                                 jnp.float32)
        p = acc_cnt[...] / n_rows
        ent = jnp.sum(p * jnp.log(p + 1e-10), axis=1, keepdims=True)
        ppl_ref[...] = jnp.exp(-ent)


def kernel(inputs, weight):
    n, dim = inputs.shape
    n_codes = weight.shape[0]
    block_rows = 2048 if n % 2048 == 0 else n
    nblk = n // block_rows

    enc, q_ste, loss, ppl = pl.pallas_call(
        functools.partial(_vq_kernel, nblk=nblk, n_rows=n,
                          n_codes=n_codes, dim=dim),
        grid=(nblk,),
        in_specs=[
            pl.BlockSpec((block_rows, dim), lambda i: (i, 0)),
            pl.BlockSpec((n_codes, dim), lambda i: (0, 0)),
        ],
        out_specs=[
            pl.BlockSpec((block_rows, n_codes), lambda i: (i, 0)),
            pl.BlockSpec((block_rows, dim), lambda i: (i, 0)),
            pl.BlockSpec((1, 1), lambda i: (0, 0)),
            pl.BlockSpec((1, 1), lambda i: (0, 0)),
        ],
        out_shape=[
            jax.ShapeDtypeStruct((n, n_codes), jnp.float32),
            jax.ShapeDtypeStruct((n, dim), jnp.float32),
            jax.ShapeDtypeStruct((1, 1), jnp.float32),
            jax.ShapeDtypeStruct((1, 1), jnp.float32),
        ],
        scratch_shapes=[
            pltpu.SMEM((1, 1), jnp.float32),
            pltpu.VMEM((1, n_codes), jnp.float32),
        ],
    )(inputs, weight)

    return (loss.reshape(()), q_ste, ppl.reshape(()), enc)


# speedup vs baseline: 1.2746x; 1.0592x over previous
import functools

import jax
import jax.numpy as jnp
from jax.experimental import pallas as pl
from jax.experimental.pallas import tpu as pltpu

_COMMITMENT_COST = 0.25


def _vq_kernel(x_ref, w_ref, enc_ref, q_ref, loss_ref, ppl_ref,
               acc_loss, acc_cnt, *, nblk, n_rows, n_codes, dim):
    i = pl.program_id(0)
    x = x_ref[...]
    w = w_ref[...]
    xsq = jnp.sum(x * x, axis=1, keepdims=True)
    wsq = jnp.sum(w * w, axis=1, keepdims=True).T
    mm2 = jax.lax.dot_general(
        2.0 * x, w, (((1,), (1,)), ((), ())),
        preferred_element_type=jnp.float32)
    d = (xsq + wsq) - mm2
    dmin = jnp.min(d, axis=1, keepdims=True)
    col = jax.lax.broadcasted_iota(
        jnp.int32, (1, d.shape[1]), 1).astype(jnp.float32)
    idx = jnp.min(jnp.where(d == dmin, col, float(n_codes)),
                  axis=1, keepdims=True)
    enc = (col == idx).astype(jnp.float32)
    enc_ref[...] = enc
    q = jax.lax.dot_general(
        enc, w, (((1,), (0,)), ((), ())),
        preferred_element_type=jnp.float32)
    diff = q - x
    q_ref[...] = x + diff
    blk_loss = jnp.sum(diff * diff)
    ones_row = jnp.ones((1, enc.shape[0]), jnp.float32)
    cnt = jax.lax.dot_general(
        ones_row, enc, (((1,), (0,)), ((), ())),
        preferred_element_type=jnp.float32)

    @pl.when(i == 0)
    def _init():
        acc_loss[0, 0] = blk_loss
        acc_cnt[...] = cnt

    @pl.when(i > 0)
    def _acc():
        acc_loss[0, 0] += blk_loss
        acc_cnt[...] += cnt

    @pl.when(i == nblk - 1)
    def _fini():
        mean_sq = acc_loss[0, 0] / (n_rows * dim)
        loss_ref[...] = jnp.full((1, 1), mean_sq + _COMMITMENT_COST * mean_sq,
                                 jnp.float32)
        p = acc_cnt[...] / n_rows
        ent = jnp.sum(p * jnp.log(p + 1e-10), axis=1, keepdims=True)
        ppl_ref[...] = jnp.exp(-ent)


def kernel(inputs, weight):
    n, dim = inputs.shape
    n_codes = weight.shape[0]
    block_rows = 2048 if n % 2048 == 0 else n
    nblk = n // block_rows

    enc, q_ste, loss, ppl = pl.pallas_call(
        functools.partial(_vq_kernel, nblk=nblk, n_rows=n,
                          n_codes=n_codes, dim=dim),
        grid=(nblk,),
        in_specs=[
            pl.BlockSpec((block_rows, dim), lambda i: (i, 0)),
            pl.BlockSpec((n_codes, dim), lambda i: (0, 0)),
        ],
        out_specs=[
            pl.BlockSpec((block_rows, n_codes), lambda i: (i, 0)),
            pl.BlockSpec((block_rows, dim), lambda i: (i, 0)),
            pl.BlockSpec((1, 1), lambda i: (0, 0)),
            pl.BlockSpec((1, 1), lambda i: (0, 0)),
        ],
        out_shape=[
            jax.ShapeDtypeStruct((n, n_codes), jnp.float32),
            jax.ShapeDtypeStruct((n, dim), jnp.float32),
            jax.ShapeDtypeStruct((1, 1), jnp.float32),
            jax.ShapeDtypeStruct((1, 1), jnp.float32),
        ],
        scratch_shapes=[
            pltpu.SMEM((1, 1), jnp.float32),
            pltpu.VMEM((1, n_codes), jnp.float32),
        ],
    )(inputs, weight)

    return (loss.reshape(()), q_ste, ppl.reshape(()), enc)
